# Initial kernel scaffold; baseline (speedup 1.0000x reference)
import jax
import jax.numpy as jnp
from jax.experimental import pallas as pl


def kernel(x, table):
    raise NotImplementedError("write your pallas kernel here")


# speedup vs baseline: 1.1080x; 1.1080x over previous
import functools

import jax
import jax.numpy as jnp
from jax import lax
from jax.experimental import pallas as pl
from jax.experimental.pallas import tpu as pltpu
from jax.experimental.pallas import tpu_sc as plsc

NUM_CORES = 2
NUM_SUBCORES = 16
NW = NUM_CORES * NUM_SUBCORES
CHUNK = 1280


@functools.lru_cache(maxsize=None)
def _build(b_flat: int, emb: int):
    b_per_w = b_flat // NW
    n_chunks = b_per_w // CHUNK
    mesh = plsc.VectorSubcoreMesh(
        core_axis_name="c", subcore_axis_name="s",
        num_cores=NUM_CORES, num_subcores=NUM_SUBCORES,
    )

    @functools.partial(
        pl.kernel,
        out_type=jax.ShapeDtypeStruct((b_flat, emb), jnp.float32),
        mesh=mesh,
        compiler_params=pltpu.CompilerParams(use_tc_tiling_on_sc=False),
        scratch_types=[
            pltpu.VMEM((CHUNK,), jnp.int32),
            pltpu.VMEM((CHUNK,), jnp.int32),
            pltpu.VMEM((CHUNK, emb), jnp.float32),
            pltpu.VMEM((CHUNK, emb), jnp.float32),
            pltpu.SemaphoreType.DMA,
            pltpu.SemaphoreType.DMA,
        ],
    )
    def emb_kernel(idx_hbm, table_hbm, out_hbm, idx_a, idx_b,
                   rows_a, rows_b, sem_a, sem_b):
        wid = lax.axis_index("s") * NUM_CORES + lax.axis_index("c")
        base = wid * b_per_w
        idx_bufs = (idx_a, idx_b)
        row_bufs = (rows_a, rows_b)
        sems = (sem_a, sem_b)

        pltpu.sync_copy(idx_hbm.at[pl.ds(base, CHUNK)], idx_a)
        prev = pltpu.async_copy(table_hbm.at[idx_a], rows_a, sem_a)
        for i in range(1, n_chunks):
            b = i % 2
            pltpu.sync_copy(idx_hbm.at[pl.ds(base + i * CHUNK, CHUNK)], idx_bufs[b])
            cur = pltpu.async_copy(table_hbm.at[idx_bufs[b]], row_bufs[b], sems[b])
            prev.wait()
            pltpu.sync_copy(
                row_bufs[1 - b], out_hbm.at[pl.ds(base + (i - 1) * CHUNK, CHUNK)]
            )
            prev = cur
        prev.wait()
        pltpu.sync_copy(
            row_bufs[(n_chunks - 1) % 2],
            out_hbm.at[pl.ds(base + (n_chunks - 1) * CHUNK, CHUNK)],
        )

    return emb_kernel


def kernel(x, table):
    b, h = x.shape
    emb = table.shape[1]
    idx_flat = x.reshape(b * h).astype(jnp.int32)
    out = _build(b * h, emb)(idx_flat, table)
    return out.reshape(b, h, emb)


# speedup vs baseline: 1.5838x; 1.4294x over previous
import functools

import jax
import jax.numpy as jnp
from jax import lax
from jax.experimental import pallas as pl
from jax.experimental.pallas import tpu as pltpu
from jax.experimental.pallas import tpu_sc as plsc

NUM_CORES = 2
NUM_SUBCORES = 16
NW = NUM_CORES * NUM_SUBCORES

HIST = 50
EMB = 32
CB = 128
CB_PER_W = 4
ROWS_CB = CB * HIST
ROWS_Q = ROWS_CB // 2
QC = 64


@functools.lru_cache(maxsize=None)
def _build(n_b: int):
    n_cb = n_b // CB
    assert n_cb == NW * CB_PER_W
    mesh = plsc.VectorSubcoreMesh(
        core_axis_name="c", subcore_axis_name="s",
        num_cores=NUM_CORES, num_subcores=NUM_SUBCORES,
    )

    @functools.partial(
        pl.kernel,
        out_type=jax.ShapeDtypeStruct((HIST, EMB // 8, n_cb, 8, CB), jnp.float32),
        mesh=mesh,
        compiler_params=pltpu.CompilerParams(
            use_tc_tiling_on_sc=False, needs_layout_passes=False,
        ),
        scratch_types=[
            pltpu.VMEM((ROWS_CB,), jnp.int32),
            pltpu.VMEM((ROWS_Q, EMB), jnp.float32),
            pltpu.VMEM((EMB // 8, 8, QC), jnp.float32),
            pltpu.VMEM((EMB // 8, 8, QC), jnp.float32),
            pltpu.SemaphoreType.DMA,
            pltpu.SemaphoreType.DMA,
            pltpu.SemaphoreType.DMA,
        ],
    )
    def emb_kernel(idx_hbm, table_hbm, out_hbm, idx_v, gbuf, stg0, stg1,
                   sem_g, sw0, sw1):
        wid = lax.axis_index("s") * NUM_CORES + lax.axis_index("c")
        stgs = (stg0, stg1)
        sems = (sw0, sw1)
        rows16 = lax.iota(jnp.int32, 16) * HIST

        def cb_body(i, carry):
            cb = wid * CB_PER_W + i
            pltpu.sync_copy(idx_hbm.at[pl.ds(cb * ROWS_CB, ROWS_CB)], idx_v)
            for q in range(2):
                pltpu.async_copy(
                    table_hbm.at[idx_v.at[pl.ds(q * ROWS_Q, ROWS_Q)]],
                    gbuf, sem_g,
                ).wait()

                def h_body(hi, carry2):
                    for t in range(2):
                        hh = hi * 2 + t
                        stg = stgs[t]
                        sem = sems[t]

                        @pl.when(hi >= 1)
                        def _wait_prev():
                            pltpu.make_async_copy(
                                stg,
                                out_hbm.at[hh - 2, :, cb, :, pl.ds(q * QC, QC)],
                                sem,
                            ).wait()

                        rows_h = rows16 + hh
                        for e in range(EMB):
                            tr, r = e // 8, e % 8
                            cols = jnp.full((16,), e, jnp.int32)
                            for g in range(QC // 16):
                                rows = rows_h + g * 16 * HIST
                                vals = plsc.load_gather(gbuf, [rows, cols])
                                stg[tr, r, pl.ds(g * 16, 16)] = vals
                        pltpu.async_copy(
                            stg,
                            out_hbm.at[hh, :, cb, :, pl.ds(q * QC, QC)],
                            sem,
                        )
                    return carry2

                lax.fori_loop(0, HIST // 2, h_body, 0)
                for t in range(2):
                    pltpu.make_async_copy(
                        stgs[t],
                        out_hbm.at[HIST - 2 + t, :, cb, :, pl.ds(q * QC, QC)],
                        sems[t],
                    ).wait()
            return carry

        lax.fori_loop(0, CB_PER_W, cb_body, 0)

    return emb_kernel


def kernel(x, table):
    b, h = x.shape
    emb = table.shape[1]
    idx_flat = x.reshape(b * h).astype(jnp.int32)
    out5 = _build(b)(idx_flat, table)
    return out5.transpose(2, 4, 0, 1, 3).reshape(b, h, emb)


# speedup vs baseline: 2.4995x; 1.5781x over previous
import functools

import jax
import jax.numpy as jnp
from jax import lax
from jax.experimental import pallas as pl
from jax.experimental.pallas import tpu as pltpu
from jax.experimental.pallas import tpu_sc as plsc

NUM_CORES = 2
NUM_SUBCORES = 16
NW = NUM_CORES * NUM_SUBCORES

HIST = 50
EMB = 32
CB = 128
CB_PER_W = 4
ROWS_CB = CB * HIST
ROWS_Q = ROWS_CB // 2
QC = 64


@functools.lru_cache(maxsize=None)
def _build(n_b: int):
    n_cb = n_b // CB
    assert n_cb == NW * CB_PER_W
    mesh = plsc.VectorSubcoreMesh(
        core_axis_name="c", subcore_axis_name="s",
        num_cores=NUM_CORES, num_subcores=NUM_SUBCORES,
    )

    @functools.partial(
        pl.kernel,
        out_type=jax.ShapeDtypeStruct((HIST, EMB // 8, n_cb, 8, CB), jnp.float32),
        mesh=mesh,
        compiler_params=pltpu.CompilerParams(
            use_tc_tiling_on_sc=False, needs_layout_passes=False,
        ),
        scratch_types=[
            pltpu.VMEM((ROWS_CB,), jnp.int32),
            pltpu.VMEM((ROWS_Q, EMB), jnp.float32),
            pltpu.VMEM((EMB // 8, 8, QC + 1), jnp.float32),
            pltpu.VMEM((EMB // 8, 8, QC + 1), jnp.float32),
            pltpu.SemaphoreType.DMA,
            pltpu.SemaphoreType.DMA,
            pltpu.SemaphoreType.DMA,
        ],
    )
    def emb_kernel(idx_hbm, table_hbm, out_hbm, idx_v, gbuf, stg0, stg1,
                   sem_g, sw0, sw1):
        wid = lax.axis_index("s") * NUM_CORES + lax.axis_index("c")
        stgs = (stg0, stg1)
        sems = (sw0, sw1)
        iota = lax.iota(jnp.int32, 16)
        tr0 = lax.shift_right_logical(iota, 3)
        tr1 = tr0 + 2
        rr = lax.bitwise_and(iota, 7)

        def cb_body(i, carry):
            cb = wid * CB_PER_W + i
            pltpu.sync_copy(idx_hbm.at[pl.ds(cb * ROWS_CB, ROWS_CB)], idx_v)
            for q in range(2):
                pltpu.async_copy(
                    table_hbm.at[idx_v.at[pl.ds(q * ROWS_Q, ROWS_Q)]],
                    gbuf, sem_g,
                ).wait()

                def h_body(hi, carry2):
                    for t in range(2):
                        hh = hi * 2 + t
                        stg = stgs[t]
                        sem = sems[t]

                        @pl.when(hi >= 1)
                        def _wait_prev():
                            pltpu.make_async_copy(
                                stg.at[:, :, pl.ds(0, QC)],
                                out_hbm.at[hh - 2, :, cb, :, pl.ds(q * QC, QC)],
                                sem,
                            ).wait()

                        for c in range(QC):
                            row = c * HIST + hh
                            cvec = jnp.broadcast_to(jnp.int32(c), (16,))
                            v0 = gbuf[row, pl.ds(0, 16)]
                            v1 = gbuf[row, pl.ds(16, 16)]
                            plsc.store_scatter(stg, [tr0, rr, cvec], v0)
                            plsc.store_scatter(stg, [tr1, rr, cvec], v1)
                        pltpu.async_copy(
                            stg.at[:, :, pl.ds(0, QC)],
                            out_hbm.at[hh, :, cb, :, pl.ds(q * QC, QC)],
                            sem,
                        )
                    return carry2

                lax.fori_loop(0, HIST // 2, h_body, 0)
                for t in range(2):
                    pltpu.make_async_copy(
                        stgs[t].at[:, :, pl.ds(0, QC)],
                        out_hbm.at[HIST - 2 + t, :, cb, :, pl.ds(q * QC, QC)],
                        sems[t],
                    ).wait()
            return carry

        lax.fori_loop(0, CB_PER_W, cb_body, 0)

    return emb_kernel


def kernel(x, table):
    b, h = x.shape
    emb = table.shape[1]
    idx_flat = x.reshape(b * h).astype(jnp.int32)
    out5 = _build(b)(idx_flat, table)
    return out5.transpose(2, 4, 0, 1, 3).reshape(b, h, emb)
